# baseline (speedup 1.0000x reference)
import functools

import jax
import jax.numpy as jnp
from jax import lax
from jax.experimental import pallas as pl
from jax.experimental.pallas import tpu as pltpu
from jax.experimental.pallas import tpu_sc as plsc

_E = 320000
_D = 128
_H = 4
_NN = 10240
_NC = 2
_NS = 16
_NW = _NC * _NS
_EPC = _E // _NW
_CW = 80
_ROWS = _EPC // _CW
_ZS = _NN // _NS
_R = 6400
_SR = _R // 128
_G = 32
_RP = _R // _G
_KP = _G * _D


def _tc_score_body(x_ref, w1bd_ref, b1t_ref, s_ref, rsel_ref, e_ref):
    h = jnp.maximum(
        jnp.dot(x_ref[...], w1bd_ref[...], preferred_element_type=jnp.float32)
        + b1t_ref[...],
        0.0,
    )
    sc = jnp.dot(h, s_ref[...], preferred_element_type=jnp.float32)
    pieces = [
        jnp.dot(rsel_ref[q], sc, preferred_element_type=jnp.float32)
        for q in range(4)
    ]
    e_ref[0] = jnp.exp(jnp.concatenate(pieces, axis=1))


def _tc_scores(x, w1, b1, w2):
    x32 = x.reshape(_E // _G, _KP)
    eye = jnp.eye(_G, dtype=jnp.float32)
    w1bd = jnp.kron(eye, w1)
    smix = jnp.kron(eye, w2)
    b1t = jnp.tile(b1, (1, _G))
    rsel = (jnp.arange(_RP)[None, None, :]
            == (4 * jnp.arange(_SR)[None, :, None]
                + jnp.arange(4)[:, None, None])).astype(jnp.float32)
    return pl.pallas_call(
        _tc_score_body,
        grid=(_E // _R,),
        in_specs=[
            pl.BlockSpec((_RP, _KP), lambda i: (i, 0)),
            pl.BlockSpec((_KP, _G * _H), lambda i: (0, 0)),
            pl.BlockSpec((1, _G * _H), lambda i: (0, 0)),
            pl.BlockSpec((_G * _H, _G), lambda i: (0, 0)),
            pl.BlockSpec((4, _SR, _RP), lambda i: (0, 0, 0)),
        ],
        out_specs=pl.BlockSpec((1, _SR, 128), lambda i: (i, 0, 0)),
        out_shape=jax.ShapeDtypeStruct((_E // _R, _SR, 128), jnp.float32),
    )(x32, w1bd, b1t, smix, rsel)


@functools.cache
def _sc_mesh():
    return plsc.VectorSubcoreMesh(core_axis_name="c", subcore_axis_name="s")


def _sc_partials(e3, idx3):
    @functools.partial(
        pl.kernel,
        out_type=jax.ShapeDtypeStruct((_NC, _NN), jnp.float32),
        mesh=_sc_mesh(),
        scratch_types=[
            pltpu.VMEM((_ROWS, _CW), jnp.float32),
            pltpu.VMEM((_ROWS, _CW), jnp.int32),
            pltpu.VMEM((_ZS,), jnp.float32),
            pltpu.VMEM_SHARED((_NN,), jnp.float32),
        ],
    )
    def k(e_hbm, idx_hbm, p_hbm, ev, iv, zv, nodesum):
        c = lax.axis_index("c")
        s = lax.axis_index("s")
        wid = c * _NS + s

        @pl.loop(0, _ZS, step=16)
        def _(t):
            zv[pl.ds(t, 16)] = jnp.zeros((16,), jnp.float32)

        pltpu.sync_copy(zv, nodesum.at[pl.ds(s * _ZS, _ZS)])

        pltpu.sync_copy(e_hbm.at[wid], ev)
        pltpu.sync_copy(idx_hbm.at[wid], iv)

        plsc.subcore_barrier()

        @pl.loop(0, _ROWS)
        def _(j):
            pltpu.sync_copy(ev.at[j], nodesum.at[iv.at[j]], add=True)

        plsc.subcore_barrier()

        pltpu.sync_copy(nodesum.at[pl.ds(s * _ZS, _ZS)], zv)
        pltpu.sync_copy(zv, p_hbm.at[c, pl.ds(s * _ZS, _ZS)])

    return k(e3, idx3)


def _sc_normalize(e3, idx3, p):
    @functools.partial(
        pl.kernel,
        out_type=jax.ShapeDtypeStruct((_NW, _ROWS, _CW), jnp.float32),
        mesh=_sc_mesh(),
        scratch_types=[
            pltpu.VMEM((_ROWS, _CW), jnp.float32),
            pltpu.VMEM((_ROWS, _CW), jnp.int32),
            pltpu.VMEM((_ROWS, _CW), jnp.float32),
            pltpu.VMEM((_ZS,), jnp.float32),
            pltpu.VMEM((_ZS,), jnp.float32),
            pltpu.VMEM_SHARED((_NN,), jnp.float32),
        ],
    )
    def k(e_hbm, idx_hbm, p_hbm, out_hbm, ev, iv, dv, pa, pb, rdenom):
        c = lax.axis_index("c")
        s = lax.axis_index("s")
        wid = c * _NS + s

        pltpu.sync_copy(p_hbm.at[0, pl.ds(s * _ZS, _ZS)], pa)
        pltpu.sync_copy(p_hbm.at[1, pl.ds(s * _ZS, _ZS)], pb)

        @pl.loop(0, _ZS, step=16)
        def _(t):
            pa[pl.ds(t, 16)] = 1.0 / (pa[pl.ds(t, 16)] + pb[pl.ds(t, 16)]
                                      + 1e-16)

        pltpu.sync_copy(pa, rdenom.at[pl.ds(s * _ZS, _ZS)])

        pltpu.sync_copy(e_hbm.at[wid], ev)
        pltpu.sync_copy(idx_hbm.at[wid], iv)

        plsc.subcore_barrier()

        @pl.loop(0, _ROWS)
        def _(j):
            pltpu.sync_copy(rdenom.at[iv.at[j]], dv.at[j])

        @pl.loop(0, _ROWS)
        def _(j):
            @pl.loop(0, _CW, step=16)
            def _(t):
                ev[j, pl.ds(t, 16)] = ev[j, pl.ds(t, 16)] * dv[j, pl.ds(t, 16)]

        pltpu.sync_copy(ev, out_hbm.at[wid])

    return k(e3, idx3, p)


def kernel(edge_latents, index, W1, b1, W2):
    e = _tc_scores(edge_latents, W1, b1.reshape(1, _H), W2)
    e3 = e.reshape(_NW, _ROWS, _CW)
    idx3 = index.reshape(_NW, _ROWS, _CW)
    p = _sc_partials(e3, idx3)
    alpha3 = _sc_normalize(e3, idx3, p)
    return alpha3.reshape(_E, 1)


# speedup vs baseline: 2.2992x; 2.2992x over previous
import functools

import jax
import jax.numpy as jnp
from jax import lax
from jax.experimental import pallas as pl
from jax.experimental.pallas import tpu as pltpu
from jax.experimental.pallas import tpu_sc as plsc

_E = 320000
_D = 128
_H = 4
_NN = 10240
_NC = 2
_NS = 16
_NW = _NC * _NS
_EPC = _E // _NW
_CW = 80
_ROWS = _EPC // _CW
_ZS = _NN // _NS
_R = 6400
_SR = _R // 128


def _tc_score_body(x_ref, w1_ref, b1_ref, w2_ref, e_ref):
    h = jnp.maximum(
        jnp.dot(x_ref[...], w1_ref[...], preferred_element_type=jnp.float32)
        + b1_ref[...],
        0.0,
    )
    s = jnp.sum(h * w2_ref[...], axis=1, keepdims=True)
    e_ref[...] = jnp.exp(s.reshape(1, _SR, 128))


def _tc_scores(x, w1, b1, w2):
    return pl.pallas_call(
        _tc_score_body,
        grid=(_E // _R,),
        in_specs=[
            pl.BlockSpec((_R, _D), lambda i: (i, 0)),
            pl.BlockSpec((_D, _H), lambda i: (0, 0)),
            pl.BlockSpec((1, _H), lambda i: (0, 0)),
            pl.BlockSpec((1, _H), lambda i: (0, 0)),
        ],
        out_specs=pl.BlockSpec((1, _SR, 128), lambda i: (i, 0, 0)),
        out_shape=jax.ShapeDtypeStruct((_E // _R, _SR, 128), jnp.float32),
    )(x, w1, b1, w2)


@functools.cache
def _sc_mesh():
    return plsc.VectorSubcoreMesh(core_axis_name="c", subcore_axis_name="s")


def _sc_partials(e3, idx3):
    @functools.partial(
        pl.kernel,
        out_type=jax.ShapeDtypeStruct((_NC, _NN), jnp.float32),
        mesh=_sc_mesh(),
        scratch_types=[
            pltpu.VMEM((_ROWS, _CW), jnp.float32),
            pltpu.VMEM((_ROWS, _CW), jnp.int32),
            pltpu.VMEM((_ZS,), jnp.float32),
            pltpu.VMEM_SHARED((_NN,), jnp.float32),
            pltpu.SemaphoreType.DMA,
            pltpu.SemaphoreType.DMA,
        ],
    )
    def k(e_hbm, idx_hbm, p_hbm, ev, iv, zv, nodesum, ldsem, scsem):
        c = lax.axis_index("c")
        s = lax.axis_index("s")
        wid = c * _NS + s

        ld_e = pltpu.async_copy(e_hbm.at[wid], ev, ldsem)
        ld_i = pltpu.async_copy(idx_hbm.at[wid], iv, ldsem)

        for t in range(0, _ZS, 16):
            zv[pl.ds(t, 16)] = jnp.zeros((16,), jnp.float32)

        pltpu.sync_copy(zv, nodesum.at[pl.ds(s * _ZS, _ZS)])

        ld_e.wait()
        ld_i.wait()

        plsc.subcore_barrier()

        for g in range(0, _ROWS, 25):
            cps = [
                pltpu.async_copy(ev.at[g + j], nodesum.at[iv.at[g + j]],
                                 scsem, add=True)
                for j in range(25)
            ]
            for cp in cps:
                cp.wait()

        plsc.subcore_barrier()

        pltpu.sync_copy(nodesum.at[pl.ds(s * _ZS, _ZS)], zv)
        pltpu.sync_copy(zv, p_hbm.at[c, pl.ds(s * _ZS, _ZS)])

    return k(e3, idx3)


def _sc_normalize(e3, idx3, p):
    @functools.partial(
        pl.kernel,
        out_type=jax.ShapeDtypeStruct((_NW, _ROWS, _CW), jnp.float32),
        mesh=_sc_mesh(),
        scratch_types=[
            pltpu.VMEM((_ROWS, _CW), jnp.float32),
            pltpu.VMEM((_ROWS, _CW), jnp.int32),
            pltpu.VMEM((_ROWS, _CW), jnp.float32),
            pltpu.VMEM((_ZS,), jnp.float32),
            pltpu.VMEM((_ZS,), jnp.float32),
            pltpu.VMEM_SHARED((_NN,), jnp.float32),
            pltpu.SemaphoreType.DMA,
            pltpu.SemaphoreType.DMA,
        ],
    )
    def k(e_hbm, idx_hbm, p_hbm, out_hbm, ev, iv, dv, pa, pb, rdenom,
          ldsem, gtsem):
        c = lax.axis_index("c")
        s = lax.axis_index("s")
        wid = c * _NS + s

        ld_a = pltpu.async_copy(p_hbm.at[0, pl.ds(s * _ZS, _ZS)], pa, ldsem)
        ld_b = pltpu.async_copy(p_hbm.at[1, pl.ds(s * _ZS, _ZS)], pb, ldsem)
        ld_e = pltpu.async_copy(e_hbm.at[wid], ev, ldsem)
        ld_i = pltpu.async_copy(idx_hbm.at[wid], iv, ldsem)

        ld_a.wait()
        ld_b.wait()

        for t in range(0, _ZS, 16):
            pa[pl.ds(t, 16)] = 1.0 / (pa[pl.ds(t, 16)] + pb[pl.ds(t, 16)]
                                      + 1e-16)

        pltpu.sync_copy(pa, rdenom.at[pl.ds(s * _ZS, _ZS)])

        ld_e.wait()
        ld_i.wait()

        plsc.subcore_barrier()

        grp = 25

        def fire(g):
            return [
                pltpu.async_copy(rdenom.at[iv.at[g + j]], dv.at[g + j], gtsem)
                for j in range(grp)
            ]

        def mul_rows(g):
            for j in range(grp):
                for t in range(0, _CW, 16):
                    ev[g + j, pl.ds(t, 16)] = (
                        ev[g + j, pl.ds(t, 16)] * dv[g + j, pl.ds(t, 16)])

        pend = fire(0)
        for g in range(0, _ROWS, grp):
            for cp in pend:
                cp.wait()
            if g + grp < _ROWS:
                pend = fire(g + grp)
            mul_rows(g)

        pltpu.sync_copy(ev, out_hbm.at[wid])

    return k(e3, idx3, p)


def kernel(edge_latents, index, W1, b1, W2):
    e = _tc_scores(edge_latents, W1, b1.reshape(1, _H), W2.reshape(1, _H))
    e3 = e.reshape(_NW, _ROWS, _CW)
    idx3 = index.reshape(_NW, _ROWS, _CW)
    p = _sc_partials(e3, idx3)
    alpha3 = _sc_normalize(e3, idx3, p)
    return alpha3.reshape(_E, 1)
